# Initial kernel scaffold; baseline (speedup 1.0000x reference)
import jax
import jax.numpy as jnp
from jax.experimental import pallas as pl


def kernel(x_protein, x_go, x_drug, params, edge_pg, edge_gp, edge_dd):
    raise NotImplementedError("write your pallas kernel here")


# speedup vs baseline: 3.3084x; 3.3084x over previous
import functools

import jax
import jax.numpy as jnp
from jax import lax
from jax.experimental import pallas as pl
from jax.experimental.pallas import tpu as pltpu
from jax.experimental.pallas import tpu_sc as plsc

_H = 128
_N = 10000
_NACC = 10240
_NW = 32
_CH = 128
_RPT = _NACC // 16
_PAD = _NACC - 1
_R = 1000
_GRID = _N // _R

_NCH_PG = 79
_NCH_DD = 40


def _mm(x, w):
    return lax.dot_general(x, w, (((1,), (1,)), ((), ())),
                           preferred_element_type=jnp.float32)


def _prelu(x, a):
    return jnp.where(x >= 0, x, a * x)


@functools.lru_cache(maxsize=None)
def _segsum_kernel(nch):
    mesh = plsc.VectorSubcoreMesh(core_axis_name="c", subcore_axis_name="s")

    @functools.partial(
        pl.kernel, mesh=mesh,
        out_type=jax.ShapeDtypeStruct((2, _NACC, _H), jnp.float32),
        scratch_types=[
            pltpu.VMEM((nch, _CH), jnp.int32),
            pltpu.VMEM((nch, _CH), jnp.int32),
            pltpu.VMEM((_CH, _H), jnp.float32),
            pltpu.VMEM_SHARED((_NACC, _H), jnp.float32),
            pltpu.SemaphoreType.DMA,
        ],
    )
    def seg(table, gidx, sidx, out, gix_v, six_v, rows_v, acc_sh, sem):
        c = lax.axis_index("c")
        s = lax.axis_index("s")
        wid = c * 16 + s
        zero16 = jnp.zeros((16,), jnp.float32)

        def zrow(r, carry):
            for g in range(_H // 16):
                rows_v[r, g * 16:(g + 1) * 16] = zero16
            return carry
        lax.fori_loop(0, _CH, zrow, 0)

        base = s * _RPT
        for k in range(_RPT // _CH):
            pltpu.sync_copy(rows_v, acc_sh.at[pl.ds(base + k * _CH, _CH)])
        pltpu.sync_copy(gidx.at[wid], gix_v)
        pltpu.sync_copy(sidx.at[wid], six_v)
        plsc.subcore_barrier()

        def body(j, carry):
            pltpu.async_copy(table.at[gix_v.at[j]], rows_v, sem).wait()
            pltpu.sync_copy(rows_v, acc_sh.at[six_v.at[j]], add=True)
            return carry
        lax.fori_loop(0, nch, body, 0)
        plsc.subcore_barrier()
        pltpu.sync_copy(acc_sh.at[pl.ds(base, _RPT)],
                        out.at[c, pl.ds(base, _RPT)])

    return seg


def _sc_segsum(table, gidx, sidx, nch):
    return _segsum_kernel(nch)(table, gidx, sidx)


@functools.lru_cache(maxsize=None)
def _counts_kernel():
    mesh = plsc.VectorSubcoreMesh(core_axis_name="c", subcore_axis_name="s")
    nchs = (_NCH_PG, _NCH_PG, _NCH_PG, _NCH_DD)

    @functools.partial(
        pl.kernel, mesh=mesh,
        out_type=jax.ShapeDtypeStruct((4, _NW, _NACC), jnp.float32),
        compiler_params=pltpu.CompilerParams(needs_layout_passes=False),
        scratch_types=[
            pltpu.VMEM((_NCH_PG, _CH), jnp.int32),
            pltpu.VMEM((_NACC,), jnp.float32),
            pltpu.VMEM((_NACC,), jnp.float32),
            pltpu.VMEM((_NACC,), jnp.float32),
            pltpu.VMEM((_NACC,), jnp.float32),
        ],
    )
    def cnts(i0, i1, i2, i3, out, idx_v, c0, c1, c2, c3):
        c = lax.axis_index("c")
        s = lax.axis_index("s")
        wid = c * 16 + s
        zero16 = jnp.zeros((16,), jnp.float32)
        ones16 = jnp.full((16,), 1.0, jnp.float32)

        for cref in (c0, c1, c2, c3):
            def zb(r, carry, cref=cref):
                cref[pl.ds(r * 16, 16)] = zero16
                return carry
            lax.fori_loop(0, _NACC // 16, zb, 0)

        for k, (ih, cref) in enumerate(zip((i0, i1, i2, i3),
                                           (c0, c1, c2, c3))):
            nch = nchs[k]
            pltpu.sync_copy(ih.at[wid], idx_v.at[pl.ds(0, nch)])

            def hb(j, carry, cref=cref):
                for g in range(_CH // 16):
                    iv = idx_v[j, g * 16:(g + 1) * 16]
                    plsc.addupdate_scatter(cref, [iv], ones16)
                return carry
            lax.fori_loop(0, nch, hb, 0)

        for k, cref in enumerate((c0, c1, c2, c3)):
            pltpu.sync_copy(cref, out.at[k, wid])

    return cnts


def _full(shape):
    return pl.BlockSpec(shape, lambda i: tuple(0 for _ in shape))


def _rows(shape):
    idx = {2: (lambda i: (i, 0)), 3: (lambda i: (0, i, 0))}[len(shape)]
    return pl.BlockSpec(shape, idx)


def _prep_body(xg_ref, pgw_ref, gplw_ref, ygo_ref, txg_ref):
    xg = xg_ref[...]
    ygo_ref[...] = _mm(xg, pgw_ref[...])
    txg_ref[...] = _mm(xg, gplw_ref[...])


def _tc_prep(x_go, pg_W, gp_Wl0):
    return pl.pallas_call(
        _prep_body,
        grid=(_GRID,),
        in_specs=[_rows((_R, 200)), _full((_H, 200)), _full((_H, 200))],
        out_specs=[_rows((_R, _H)), _rows((_R, _H))],
        out_shape=[jax.ShapeDtypeStruct((_N, _H), jnp.float32)] * 2,
    )(x_go, pg_W, gp_Wl0)


def _main_body(xprot, xdrug, s0, ca, pe_W, pc_W, w1e, w1c, w1g, w1p, gw2,
               fw1, fw2, dw1, dw2, pgl0, ddl0,
               b_pe, b_pc, b_pg, b_g1, b_g2, b_f1, b_f2, b_d1, b_d2, alph,
               xp_o, xd_o, txp_o, txd_o):
    x = xprot[...]
    a = alph[...]
    h_esm = _prelu(_mm(x[:, :2048], pe_W[...]) + b_pe[...], a[:, 0:1])
    h_cath = _prelu(_mm(x[:, 2048:], pc_W[...]) + b_pc[...], a[:, 1:2])
    cnt = jnp.sum(ca[...], axis=1, keepdims=True)
    ssum = s0[0] + s0[1]
    present = (cnt > 0).astype(jnp.float32)
    h_go = _prelu(ssum / jnp.maximum(cnt, 1.0) + b_pg[...], a[:, 2:3])
    g = (_mm(h_esm, w1e[...]) + _mm(h_cath, w1c[...]) + _mm(h_go, w1g[...])
         + present * w1p[...] + b_g1[...])
    g = _prelu(g, a[:, 3:4])
    logits = _mm(g, gw2[...]) + b_g2[...]
    m = jnp.max(logits, axis=-1, keepdims=True)
    e = jnp.exp(logits - m)
    w = e / jnp.sum(e, axis=-1, keepdims=True)
    h_mix = w[:, 0:1] * h_esm + w[:, 1:2] * h_cath + w[:, 2:3] * h_go
    t = _prelu(_mm(h_mix, fw1[...]) + b_f1[...], a[:, 4:5])
    xp = _prelu(_mm(t, fw2[...]) + b_f2[...], a[:, 5:6])
    td = _prelu(_mm(xdrug[...], dw1[...]) + b_d1[...], a[:, 6:7])
    xd = _prelu(_mm(td, dw2[...]) + b_d2[...], a[:, 7:8])
    xp_o[...] = xp
    xd_o[...] = xd
    txp_o[...] = _mm(xp, pgl0[...])
    txd_o[...] = _mm(xd, ddl0[...])


def _tc_main(xprot, xdrug, S0, cA, p):
    w1 = p['gate_W1']
    args = (xprot, xdrug, S0, cA,
            p['pe_W'], p['pc_W'],
            w1[:, 0:128], w1[:, 128:256], w1[:, 256:384], w1[:, 384:385].T,
            p['gate_W2'], p['fuse_W1'], p['fuse_W2'],
            p['drug_W1'], p['drug_W2'], p['c0_pg_Wl'], p['c0_dd_Wl'],
            p['pe_b'].reshape(1, _H), p['pc_b'].reshape(1, _H),
            p['pg_b'].reshape(1, _H), p['gate_b1'].reshape(1, _H),
            p['gate_b2'].reshape(1, 3), p['fuse_b1'].reshape(1, _H),
            p['fuse_b2'].reshape(1, _H), p['drug_b1'].reshape(1, _H),
            p['drug_b2'].reshape(1, _H),
            jnp.stack([p['pe_a'], p['pc_a'], p['pg_a'], p['gate_a'],
                       p['fuse_a1'], p['fuse_a2'], p['drug_a1'], p['drug_a2']]
                      ).reshape(1, 8))
    in_specs = [
        _rows((_R, 2816)), _rows((_R, 512)), _rows((2, _R, _H)),
        _rows((_R, _NW)),
        _full((_H, 2048)), _full((_H, 768)),
        _full((_H, _H)), _full((_H, _H)), _full((_H, _H)), _full((1, _H)),
        _full((3, _H)), _full((_H, _H)), _full((_H, _H)),
        _full((_H, 512)), _full((_H, _H)), _full((_H, _H)), _full((_H, _H)),
        _full((1, _H)), _full((1, _H)), _full((1, _H)), _full((1, _H)),
        _full((1, 3)), _full((1, _H)), _full((1, _H)), _full((1, _H)),
        _full((1, _H)), _full((1, 8)),
    ]
    return pl.pallas_call(
        _main_body,
        grid=(_GRID,),
        in_specs=in_specs,
        out_specs=[_rows((_R, _H))] * 4,
        out_shape=[jax.ShapeDtypeStruct((_N, _H), jnp.float32)] * 4,
    )(*args)


def _make_combine_body(goin, normalize, has_next, is_last):
    def body(*refs):
        (sg, sp, sd, cg, cp, cd, xp, xg, xd,
         wr_pg, wr_gp, wr_dd, b_pg, b_gp, b_dd, alph) = refs[:16]
        k = 16
        if has_next:
            nl_pg, nl_gp, nl_dd = refs[k:k + 3]
            k += 3
        if is_last:
            dres = refs[k]
            k += 1
        outs = refs[k:]
        a = alph[...]

        def seg(sref, cref):
            cnt = jnp.sum(cref[...], axis=1, keepdims=True)
            return (sref[0] + sref[1]) / jnp.maximum(cnt, 1.0)

        new_go = seg(sg, cg) + b_pg[...] + _mm(xg[...], wr_pg[...])
        new_p = seg(sp, cp) + b_gp[...] + _mm(xp[...], wr_gp[...])
        new_d = seg(sd, cd) + b_dd[...] + _mm(xd[...], wr_dd[...])
        if normalize:
            def nrm(v):
                n = jnp.sqrt(jnp.sum(v * v, axis=-1, keepdims=True))
                return v / jnp.maximum(n, 1e-12)
            new_go, new_p, new_d = nrm(new_go), nrm(new_p), nrm(new_d)
        xp_n = _prelu(new_p, a)
        xg_n = _prelu(new_go, a)
        xd_n = _prelu(new_d, a)
        if is_last:
            xd_n = xd_n + dres[...]
        outs[0][...] = xp_n
        outs[1][...] = xg_n
        outs[2][...] = xd_n
        if has_next:
            outs[3][...] = _mm(xp_n, nl_pg[...])
            outs[4][...] = _mm(xg_n, nl_gp[...])
            outs[5][...] = _mm(xd_n, nl_dd[...])
    return body


def _tc_combine(l, Sg, Sp, Sd, cg, cp, cd, xp, xg, xd, p, dres=None):
    goin = 200 if l == 0 else _H
    normalize = l > 0
    has_next = l < 2
    is_last = l == 2
    args = [Sg, Sp, Sd, cg, cp, cd, xp, xg, xd,
            p['c%d_pg_Wr' % l], p['c%d_gp_Wr' % l], p['c%d_dd_Wr' % l],
            p['c%d_pg_bl' % l].reshape(1, _H),
            p['c%d_gp_bl' % l].reshape(1, _H),
            p['c%d_dd_bl' % l].reshape(1, _H),
            p['act_a%d' % l].reshape(1, 1)]
    in_specs = [_rows((2, _R, _H))] * 3 + [_rows((_R, _NW))] * 3 + [
        _rows((_R, _H)), _rows((_R, goin)), _rows((_R, _H)),
        _full((_H, goin)), _full((_H, _H)), _full((_H, _H)),
        _full((1, _H)), _full((1, _H)), _full((1, _H)), _full((1, 1))]
    n_out = 3
    if has_next:
        nl = l + 1
        args += [p['c%d_pg_Wl' % nl], p['c%d_gp_Wl' % nl], p['c%d_dd_Wl' % nl]]
        in_specs += [_full((_H, _H))] * 3
        n_out = 6
    if is_last:
        args.append(dres)
        in_specs.append(_rows((_R, _H)))
    return pl.pallas_call(
        _make_combine_body(goin, normalize, has_next, is_last),
        grid=(_GRID,),
        in_specs=in_specs,
        out_specs=[_rows((_R, _H))] * n_out,
        out_shape=[jax.ShapeDtypeStruct((_N, _H), jnp.float32)] * n_out,
    )(*args)


def _edge_layout(a, padval, nch):
    tot = _NW * nch * _CH
    a = a.astype(jnp.int32)
    a = jnp.concatenate([a, jnp.full((tot - a.shape[0],), padval, jnp.int32)])
    return a.reshape(_NW, nch, _CH)


def _cnt_t(counts_k):
    return counts_k.T


def kernel(x_protein, x_go, x_drug, params, edge_pg, edge_gp, edge_dd):
    p = params
    pg0g = _edge_layout(edge_pg[0], 0, _NCH_PG)
    pg0s = _edge_layout(edge_pg[0], _PAD, _NCH_PG)
    pg1g = _edge_layout(edge_pg[1], 0, _NCH_PG)
    pg1s = _edge_layout(edge_pg[1], _PAD, _NCH_PG)
    gp0g = _edge_layout(edge_gp[0], 0, _NCH_PG)
    gp1s = _edge_layout(edge_gp[1], _PAD, _NCH_PG)
    dd0g = _edge_layout(edge_dd[0], 0, _NCH_DD)
    dd1s = _edge_layout(edge_dd[1], _PAD, _NCH_DD)

    counts = _counts_kernel()(pg0s, pg1s, gp1s, dd1s)
    cA = _cnt_t(counts[0])
    cPG = _cnt_t(counts[1])
    cGP = _cnt_t(counts[2])
    cDD = _cnt_t(counts[3])

    y_go, tx_g0 = _tc_prep(x_go, p['pg_W'], p['c0_gp_Wl'])
    S0 = _sc_segsum(y_go, pg1g, pg0s, _NCH_PG)
    xp, xd, txp, txd = _tc_main(x_protein, x_drug, S0, cA, p)
    drug_res = xd
    xg = x_go
    txg = tx_g0
    for l in range(3):
        Sg = _sc_segsum(txp, pg0g, pg1s, _NCH_PG)
        Sp = _sc_segsum(txg, gp0g, gp1s, _NCH_PG)
        Sd = _sc_segsum(txd, dd0g, dd1s, _NCH_DD)
        if l < 2:
            xp, xg, xd, txp, txg, txd = _tc_combine(
                l, Sg, Sp, Sd, cPG, cGP, cDD, xp, xg, xd, p)
        else:
            xp, xg, xd = _tc_combine(
                l, Sg, Sp, Sd, cPG, cGP, cDD, xp, xg, xd, p, dres=drug_res)
    return xp, xg, xd
